# baseline (speedup 1.0000x reference)
import functools

import jax
import jax.numpy as jnp
from jax import lax
from jax.experimental import pallas as pl
from jax.experimental.pallas import tpu as pltpu
from jax.experimental.pallas import tpu_sc as plsc

_NC = 2
_NS = 16
_NW = _NC * _NS
_LANES = 16


def _tc_add_body(x_ref, pe_ref, o_ref):
    o_ref[...] = x_ref[...] + pe_ref[...][None, :, :]


def _tc_part(x, pe, nb):
    B, S, D = x.shape
    SBLK = 512
    return pl.pallas_call(
        _tc_add_body,
        grid=(S // SBLK,),
        in_specs=[
            pl.BlockSpec((nb, SBLK, D), lambda i: (0, i, 0)),
            pl.BlockSpec((SBLK, D), lambda i: (i, 0)),
        ],
        out_specs=pl.BlockSpec((nb, SBLK, D), lambda i: (0, i, 0)),
        out_shape=jax.ShapeDtypeStruct((nb, S, D), x.dtype),
    )(x, pe)


def _make_sc_part(B, S, D, b0):
    NB = B - b0
    SPW = S // _NW
    CS = 16
    NVREG = CS * D // _LANES
    VPR = D // _LANES
    NCH = SPW // CS
    NSTEP = NCH * NB

    mesh = plsc.VectorSubcoreMesh(core_axis_name="c", subcore_axis_name="s")

    @functools.partial(
        pl.kernel,
        mesh=mesh,
        out_type=jax.ShapeDtypeStruct((NB, S, D), jnp.float32),
        scratch_types=[
            pltpu.VMEM((4, CS, D), jnp.float32),
            pltpu.VMEM((2, CS, D), jnp.float32),
            [pltpu.SemaphoreType.DMA] * 4,
            [pltpu.SemaphoreType.DMA] * 4,
            [pltpu.SemaphoreType.DMA] * 2,
        ],
    )
    def sc_kernel(x_hbm, pe_hbm, out_hbm, xbuf, pbuf, sin, sout, spe):
        wid = lax.axis_index("s") * _NC + lax.axis_index("c")
        s0 = wid * SPW

        in_h = [None] * NSTEP
        out_h = [None] * NSTEP
        pe_h = [None] * NCH

        def rows(t):
            k, b = divmod(t, NB)
            return k, b, s0 + k * CS

        def start_in(t):
            k, b, lo = rows(t)
            j = t % 4
            in_h[t] = pltpu.async_copy(
                x_hbm.at[b0 + b, pl.ds(lo, CS), :], xbuf.at[j], sin[j])

        def start_pe(k):
            pe_h[k] = pltpu.async_copy(
                pe_hbm.at[pl.ds(s0 + k * CS, CS), :], pbuf.at[k % 2],
                spe[k % 2])

        start_pe(0)
        start_in(0)
        start_in(1)
        for t in range(NSTEP):
            k, b, lo = rows(t)
            j = t % 4
            if b == 0:
                if k + 1 < NCH:
                    start_pe(k + 1)
                pe_h[k].wait()
            in_h[t].wait()

            @plsc.parallel_loop(0, NVREG, 1, unroll=8)
            def _add(i):
                r = lax.shift_right_logical(i, 6)
                c = pl.multiple_of(
                    lax.shift_left(lax.bitwise_and(i, VPR - 1), 4), _LANES)
                xbuf[j, r, pl.ds(c, _LANES)] = (
                    xbuf[j, r, pl.ds(c, _LANES)]
                    + pbuf[k % 2, r, pl.ds(c, _LANES)])

            out_h[t] = pltpu.async_copy(
                xbuf.at[j], out_hbm.at[b, pl.ds(lo, CS), :], sout[j])
            if t >= 2:
                out_h[t - 2].wait()
            if t + 2 < NSTEP:
                start_in(t + 2)
        out_h[NSTEP - 2].wait()
        out_h[NSTEP - 1].wait()

    return sc_kernel


def kernel(x, pos_emb):
    B, S, D = x.shape
    pe = pos_emb[:S]
    b0 = B - 1
    tc_out = _tc_part(x, pe, b0)
    sc = _make_sc_part(B, S, D, b0)
    sc_out = sc(x, pe)
    return jnp.concatenate([tc_out, sc_out], axis=0)


# speedup vs baseline: 1.4446x; 1.4446x over previous
import functools

import jax
import jax.numpy as jnp
from jax import lax
from jax.experimental import pallas as pl
from jax.experimental.pallas import tpu as pltpu
from jax.experimental.pallas import tpu_sc as plsc

_NC = 2
_NS = 16
_NW = _NC * _NS
_LANES = 16


def _make_sc_kernel(B, S, D):
    SPW = S // _NW
    CS = 8
    NVREG = CS * D // _LANES
    VPR = D // _LANES
    NCH = SPW // CS
    NSET = 3

    mesh = plsc.VectorSubcoreMesh(core_axis_name="c", subcore_axis_name="s")

    @functools.partial(
        pl.kernel,
        mesh=mesh,
        out_type=jax.ShapeDtypeStruct((B, S, D), jnp.float32),
        scratch_types=[
            pltpu.VMEM((NSET, B, CS, D), jnp.float32),
            pltpu.VMEM((NSET, CS, D), jnp.float32),
            [pltpu.SemaphoreType.DMA] * (NSET * B),
            [pltpu.SemaphoreType.DMA] * (NSET * B),
            [pltpu.SemaphoreType.DMA] * NSET,
        ],
    )
    def sc_kernel(x_hbm, pe_hbm, out_hbm, xbuf, pbuf, sin, sout, spe):
        wid = lax.axis_index("s") * _NC + lax.axis_index("c")
        s0 = wid * SPW

        in_h = [[None] * B for _ in range(NCH)]
        out_h = [[None] * B for _ in range(NCH)]
        pe_h = [None] * NCH

        def start_in(k):
            p = k % NSET
            lo = s0 + k * CS
            for b in range(B):
                in_h[k][b] = pltpu.async_copy(
                    x_hbm.at[b, pl.ds(lo, CS), :], xbuf.at[p, b],
                    sin[p * B + b])

        def start_pe(k):
            pe_h[k] = pltpu.async_copy(
                pe_hbm.at[pl.ds(s0 + k * CS, CS), :], pbuf.at[k % NSET],
                spe[k % NSET])

        start_pe(0)
        start_in(0)
        start_pe(1)
        start_in(1)
        for k in range(NCH):
            p = k % NSET
            lo = s0 + k * CS
            pe_h[k].wait()
            for b in range(B):
                in_h[k][b].wait()

            @plsc.parallel_loop(0, NVREG, 1, unroll=4)
            def _add(i):
                r = lax.shift_right_logical(i, 6)
                c = pl.multiple_of(
                    lax.shift_left(lax.bitwise_and(i, VPR - 1), 4), _LANES)
                pv = pbuf[p, r, pl.ds(c, _LANES)]
                for b in range(B):
                    xbuf[p, b, r, pl.ds(c, _LANES)] = (
                        xbuf[p, b, r, pl.ds(c, _LANES)] + pv)

            for b in range(B):
                out_h[k][b] = pltpu.async_copy(
                    xbuf.at[p, b], out_hbm.at[b, pl.ds(lo, CS), :],
                    sout[p * B + b])
            if k >= 1:
                for b in range(B):
                    out_h[k - 1][b].wait()
            if k + 2 < NCH:
                start_pe(k + 2)
                start_in(k + 2)
        for b in range(B):
            out_h[NCH - 1][b].wait()

    return sc_kernel


def kernel(x, pos_emb):
    B, S, D = x.shape
    pe = pos_emb[:S]
    sc = _make_sc_kernel(B, S, D)
    return sc(x, pe)
